# baseline (speedup 1.0000x reference)
import functools

import jax
import jax.numpy as jnp
from jax import lax
from jax.experimental import pallas as pl
from jax.experimental.pallas import tpu as pltpu
from jax.experimental.pallas import tpu_sc as plsc

_D = 128
_NW = 32
_BN = 1000
_BE = 2000


def _chunk_rows(epw):
    for k in (80, 40, 16, 8):
        if epw % k == 0:
            return k
    raise ValueError(epw)


def _silu(v):
    return v * jax.nn.sigmoid(v)


def _ln(h, g, b):
    mu = jnp.mean(h, axis=-1, keepdims=True)
    var = jnp.mean((h - mu) ** 2, axis=-1, keepdims=True)
    return (h - mu) * lax.rsqrt(var + 1e-5) * g + b


def _pq_body(x_ref, wa_ref, wb_ref, b1_ref, p_ref, q_ref):
    xv = x_ref[...]
    p_ref[...] = jnp.dot(xv, wa_ref[...], preferred_element_type=jnp.float32) + b1_ref[...]
    q_ref[...] = jnp.dot(xv, wb_ref[...], preferred_element_type=jnp.float32)


def _pq_call(x, wa, wb, b1):
    n = x.shape[0]
    full = lambda i: (0, 0)
    return pl.pallas_call(
        _pq_body,
        grid=(n // _BN,),
        in_specs=[
            pl.BlockSpec((_BN, _D), lambda i: (i, 0)),
            pl.BlockSpec((_D, _D), full),
            pl.BlockSpec((_D, _D), full),
            pl.BlockSpec((1, _D), full),
        ],
        out_specs=[pl.BlockSpec((_BN, _D), lambda i: (i, 0))] * 2,
        out_shape=[jax.ShapeDtypeStruct((n, _D), jnp.float32)] * 2,
    )(x, wa, wb, b1)


def _edge_body(real_e, gd_ref, gs_ref, ea_ref, w1c_ref, w2_ref, b2_ref,
               g_ref, be_ref, out_ref):
    ea = ea_ref[...]
    h = gd_ref[...] + gs_ref[...] + jnp.dot(ea, w1c_ref[...], preferred_element_type=jnp.float32)
    h = _silu(h)
    h = _silu(jnp.dot(h, w2_ref[...], preferred_element_type=jnp.float32) + b2_ref[...])
    en = _ln(h, g_ref[...], be_ref[...]) + ea
    if real_e is not None:
        rid = pl.program_id(0) * _BE + lax.broadcasted_iota(jnp.int32, (_BE, 1), 0)
        en = jnp.where(rid < real_e, en, 0.0)
    out_ref[...] = en


def _edge_call(gd, gs, ea, w1c, w2, b2, g, be, real_e=None):
    e = ea.shape[0]
    full = lambda i: (0, 0)
    row = lambda i: (i, 0)
    return pl.pallas_call(
        functools.partial(_edge_body, real_e),
        grid=(e // _BE,),
        in_specs=[
            pl.BlockSpec((_BE, _D), row),
            pl.BlockSpec((_BE, _D), row),
            pl.BlockSpec((_BE, _D), row),
            pl.BlockSpec((_D, _D), full),
            pl.BlockSpec((_D, _D), full),
            pl.BlockSpec((1, _D), full),
            pl.BlockSpec((1, _D), full),
            pl.BlockSpec((1, _D), full),
        ],
        out_specs=pl.BlockSpec((_BE, _D), row),
        out_shape=jax.ShapeDtypeStruct((e, _D), jnp.float32),
    )(gd, gs, ea, w1c, w2, b2, g, be)


def _node_body_next(x_ref, p0_ref, p1_ref, p2_ref, p3_ref, wa_ref, wb_ref,
                    b1_ref, w2_ref, b2_ref, g_ref, be_ref, nwa_ref, nwb_ref,
                    nb1_ref, xo_ref, po_ref, qo_ref):
    xv = x_ref[...]
    agg = (p0_ref[...] + p1_ref[...]) + (p2_ref[...] + p3_ref[...])
    h = (jnp.dot(xv, wa_ref[...], preferred_element_type=jnp.float32)
         + jnp.dot(agg, wb_ref[...], preferred_element_type=jnp.float32)
         + b1_ref[...])
    h = _silu(h)
    h = _silu(jnp.dot(h, w2_ref[...], preferred_element_type=jnp.float32) + b2_ref[...])
    xn = _ln(h, g_ref[...], be_ref[...]) + xv
    xo_ref[...] = xn
    po_ref[...] = jnp.dot(xn, nwa_ref[...], preferred_element_type=jnp.float32) + nb1_ref[...]
    qo_ref[...] = jnp.dot(xn, nwb_ref[...], preferred_element_type=jnp.float32)


def _node_body_last(x_ref, p0_ref, p1_ref, p2_ref, p3_ref, wa_ref, wb_ref,
                    b1_ref, w2_ref, b2_ref, g_ref, be_ref, xo_ref):
    xv = x_ref[...]
    agg = (p0_ref[...] + p1_ref[...]) + (p2_ref[...] + p3_ref[...])
    h = (jnp.dot(xv, wa_ref[...], preferred_element_type=jnp.float32)
         + jnp.dot(agg, wb_ref[...], preferred_element_type=jnp.float32)
         + b1_ref[...])
    h = _silu(h)
    h = _silu(jnp.dot(h, w2_ref[...], preferred_element_type=jnp.float32) + b2_ref[...])
    xo_ref[...] = _ln(h, g_ref[...], be_ref[...]) + xv


def _node_call(x, partials_a, partials_b, wa, wb, b1, w2, b2, g, be, nxt):
    n = x.shape[0]
    nb = n // _BN
    full = lambda i: (0, 0)
    row = lambda i: (i, 0)
    vec = pl.BlockSpec((1, _D), full)
    mat = pl.BlockSpec((_D, _D), full)
    blk = pl.BlockSpec((_BN, _D), row)
    shifted = pl.BlockSpec((_BN, _D), lambda i: (i + nb, 0))
    in_specs = [
        blk,
        blk, shifted,
        blk, shifted,
        mat, mat, vec, mat, vec, vec, vec,
    ]
    args = [x, partials_a, partials_a, partials_b, partials_b,
            wa, wb, b1, w2, b2, g, be]
    if nxt is None:
        return pl.pallas_call(
            _node_body_last,
            grid=(nb,),
            in_specs=in_specs,
            out_specs=blk,
            out_shape=jax.ShapeDtypeStruct((n, _D), jnp.float32),
        )(*args)
    nwa, nwb, nb1 = nxt
    return pl.pallas_call(
        _node_body_next,
        grid=(nb,),
        in_specs=in_specs + [mat, mat, vec],
        out_specs=[blk] * 3,
        out_shape=[jax.ShapeDtypeStruct((n, _D), jnp.float32)] * 3,
    )(*args, nwa, nwb, nb1)


_NB = 5
_SNB = 3


def _gather_call(p, q, dst, src):
    n = p.shape[0]
    e = dst.shape[0]
    epw = e // _NW
    _K = _chunk_rows(epw)
    nchunk = epw // _K
    mesh = plsc.VectorSubcoreMesh(core_axis_name="c", subcore_axis_name="s")

    @functools.partial(
        pl.kernel,
        out_type=[jax.ShapeDtypeStruct((e, _D), jnp.float32)] * 2,
        mesh=mesh,
        scratch_types=[
            pltpu.VMEM((epw,), jnp.int32),
            pltpu.VMEM((epw,), jnp.int32),
            pltpu.VMEM((_NB, _K, _D), jnp.float32),
            pltpu.VMEM((_NB, _K, _D), jnp.float32),
            pltpu.SemaphoreType.DMA((_NB,)),
            pltpu.SemaphoreType.DMA((_NB,)),
            pltpu.SemaphoreType.DMA,
        ],
    )
    def k(p_hbm, q_hbm, dst_hbm, src_hbm, gd_hbm, gs_hbm,
          idxd, idxs, bufd, bufs, semg, semo, semi):
        wid = lax.axis_index("s") * 2 + lax.axis_index("c")
        base = wid * epw
        pltpu.async_copy(dst_hbm.at[pl.ds(base, epw)], idxd, semi).wait()
        pltpu.async_copy(src_hbm.at[pl.ds(base, epw)], idxs, semi).wait()

        def start_gather(c, b):
            off = c * _K
            pltpu.async_copy(p_hbm.at[idxd.at[pl.ds(off, _K)]], bufd.at[b], semg.at[b])
            pltpu.async_copy(q_hbm.at[idxs.at[pl.ds(off, _K)]], bufs.at[b], semg.at[b])

        def wait_gather(c, b):
            off = c * _K
            pltpu.make_async_copy(p_hbm.at[idxd.at[pl.ds(off, _K)]], bufd.at[b], semg.at[b]).wait()
            pltpu.make_async_copy(q_hbm.at[idxs.at[pl.ds(off, _K)]], bufs.at[b], semg.at[b]).wait()

        def start_store(c, b):
            off = base + c * _K
            pltpu.async_copy(bufd.at[b], gd_hbm.at[pl.ds(off, _K)], semo.at[b])
            pltpu.async_copy(bufs.at[b], gs_hbm.at[pl.ds(off, _K)], semo.at[b])

        def wait_store(c, b):
            off = base + c * _K
            pltpu.make_async_copy(bufd.at[b], gd_hbm.at[pl.ds(off, _K)], semo.at[b]).wait()
            pltpu.make_async_copy(bufs.at[b], gs_hbm.at[pl.ds(off, _K)], semo.at[b]).wait()

        start_gather(0, 0)

        def body(c, carry):
            b = lax.rem(c, _NB)
            nc = c + 1
            bn = lax.rem(nc, _NB)

            @pl.when(nc < nchunk)
            def _():
                @pl.when(nc >= _NB)
                def _():
                    wait_store(nc - _NB, bn)
                start_gather(nc, bn)

            wait_gather(c, b)
            start_store(c, b)
            return carry

        lax.fori_loop(0, nchunk, body, 0)
        for t in range(_NB):
            c = nchunk - _NB + t
            wait_store(c, c % _NB)

    return k(p, q, dst, src)


def _scatter_call(en, dst, n):
    e = dst.shape[0]
    epw = e // _NW
    _K = _chunk_rows(epw)
    nchunk = epw // _K
    stripe = 624
    rem = n - 16 * stripe
    zb = 16
    mesh = plsc.VectorSubcoreMesh(core_axis_name="c", subcore_axis_name="s")

    @functools.partial(
        pl.kernel,
        out_type=jax.ShapeDtypeStruct((2 * n, _D), jnp.float32),
        mesh=mesh,
        scratch_types=[
            pltpu.VMEM((_SNB, _K), jnp.int32),
            pltpu.VMEM((_SNB, _K, _D), jnp.float32),
            pltpu.VMEM((zb, _D), jnp.float32),
            pltpu.VMEM_SHARED((n, _D), jnp.float32),
            pltpu.SemaphoreType.DMA((_SNB,)),
            pltpu.SemaphoreType.DMA((_SNB,)),
            pltpu.SemaphoreType.DMA,
        ],
    )
    def k(en_hbm, dst_hbm, out_hbm, idxb, rows, zbuf, acc, semr, semw, sem):
        cid = lax.axis_index("c")
        sid = lax.axis_index("s")
        wid = sid * 2 + cid
        base = wid * epw

        zv = jnp.zeros((16,), jnp.float32)

        def zrow(r, carry):
            for j in range(_D // 16):
                zbuf[r, pl.ds(j * 16, 16)] = zv
            return carry

        lax.fori_loop(0, zb, zrow, 0)
        my_off = sid * stripe
        nzcopy = (stripe // zb) + jnp.where(sid == 15, 1, 0)

        def zcopy(t, carry):
            pltpu.async_copy(zbuf, acc.at[pl.ds(my_off + t * zb, zb)], sem).wait()
            return carry

        lax.fori_loop(0, nzcopy, zcopy, 0)
        plsc.subcore_barrier()

        def stage_and_load(c, b):
            off = base + c * _K
            pltpu.async_copy(dst_hbm.at[pl.ds(off, _K)], idxb.at[b], semr.at[b])
            pltpu.async_copy(en_hbm.at[pl.ds(off, _K)], rows.at[b], semr.at[b])

        def wait_load(c, b):
            off = base + c * _K
            pltpu.make_async_copy(dst_hbm.at[pl.ds(off, _K)], idxb.at[b], semr.at[b]).wait()
            pltpu.make_async_copy(en_hbm.at[pl.ds(off, _K)], rows.at[b], semr.at[b]).wait()

        def start_scatter(b):
            pltpu.async_copy(rows.at[b], acc.at[idxb.at[b]], semw.at[b], add=True)

        def wait_scatter(b):
            pltpu.make_async_copy(rows.at[b], acc.at[idxb.at[b]], semw.at[b]).wait()

        stage_and_load(0, 0)

        def body(c, carry):
            b = lax.rem(c, _SNB)
            nc = c + 1
            bn = lax.rem(nc, _SNB)

            @pl.when(nc < nchunk)
            def _():
                @pl.when(nc >= _SNB)
                def _():
                    wait_scatter(bn)
                stage_and_load(nc, bn)

            wait_load(c, b)
            start_scatter(b)
            return carry

        lax.fori_loop(0, nchunk, body, 0)
        for t in range(_SNB):
            wait_scatter((nchunk - _SNB + t) % _SNB)
        plsc.subcore_barrier()
        pltpu.async_copy(acc.at[pl.ds(my_off, stripe)],
                         out_hbm.at[pl.ds(cid * n + my_off, stripe)], sem).wait()
        @pl.when(sid == 15)
        def _():
            pltpu.async_copy(acc.at[pl.ds(16 * stripe, rem)],
                             out_hbm.at[pl.ds(cid * n + 16 * stripe, rem)], sem).wait()

    return k(en, dst)


def kernel(x, edge_index, edge_attr, params):
    n = x.shape[0]
    e = edge_index.shape[1]
    h = e // 2
    align = _NW * 80
    hp = -(-h // align) * align
    pad = hp - h

    def pad_idx(v):
        return jnp.concatenate([v, jnp.arange(pad, dtype=v.dtype) % n])

    src = (edge_index[0, :h], edge_index[0, h:])
    dst = (edge_index[1, :h], edge_index[1, h:])
    srcp = (pad_idx(src[0]), pad_idx(src[1]))
    dstp = (pad_idx(dst[0]), pad_idx(dst[1]))
    ea = (edge_attr[:h], edge_attr[h:])

    def split_edge(p):
        w1 = p["edge"]["W1"]
        return (w1[:_D], w1[_D:2 * _D], p["edge"]["b1"].reshape(1, _D))

    wa0, wb0, b10 = split_edge(params[0])
    pcur, qcur = _pq_call(x, wa0, wb0, b10)

    for li, p in enumerate(params):
        pe, pn = p["edge"], p["node"]
        ew = (pe["W1"][2 * _D:], pe["W2"], pe["b2"].reshape(1, _D),
              pe["g"].reshape(1, _D), pe["be"].reshape(1, _D))
        en = [None, None]
        partials = [None, None]
        for half in range(2):
            gd, gs = _gather_call(pcur, qcur, dstp[half], srcp[half])
            en[half] = _edge_call(gd, gs, ea[half], *ew)
            partials[half] = _scatter_call(en[half], dst[half], n)
        nxt = None if li == len(params) - 1 else split_edge(params[li + 1])
        res = _node_call(x, partials[0], partials[1],
                         pn["W1"][:_D], pn["W1"][_D:], pn["b1"].reshape(1, _D),
                         pn["W2"], pn["b2"].reshape(1, _D),
                         pn["g"].reshape(1, _D), pn["be"].reshape(1, _D), nxt)
        if nxt is None:
            x = res
        else:
            x, pcur, qcur = res
        ea = (en[0], en[1])
    return x


# speedup vs baseline: 1.0007x; 1.0007x over previous
import functools

import jax
import jax.numpy as jnp
from jax import lax
from jax.experimental import pallas as pl
from jax.experimental.pallas import tpu as pltpu
from jax.experimental.pallas import tpu_sc as plsc

_D = 128
_NW = 32
_BN = 1000
_BE = 2000


def _chunk_rows(epw):
    for k in (80, 40, 16, 8):
        if epw % k == 0:
            return k
    raise ValueError(epw)


def _silu(v):
    return v * jax.nn.sigmoid(v)


def _ln(h, g, b):
    mu = jnp.mean(h, axis=-1, keepdims=True)
    var = jnp.mean((h - mu) ** 2, axis=-1, keepdims=True)
    return (h - mu) * lax.rsqrt(var + 1e-5) * g + b


def _pq_body(x_ref, wa_ref, wb_ref, b1_ref, p_ref, q_ref):
    xv = x_ref[...]
    p_ref[...] = jnp.dot(xv, wa_ref[...], preferred_element_type=jnp.float32) + b1_ref[...]
    q_ref[...] = jnp.dot(xv, wb_ref[...], preferred_element_type=jnp.float32)


def _pq_call(x, wa, wb, b1):
    n = x.shape[0]
    full = lambda i: (0, 0)
    return pl.pallas_call(
        _pq_body,
        grid=(n // _BN,),
        in_specs=[
            pl.BlockSpec((_BN, _D), lambda i: (i, 0)),
            pl.BlockSpec((_D, _D), full),
            pl.BlockSpec((_D, _D), full),
            pl.BlockSpec((1, _D), full),
        ],
        out_specs=[pl.BlockSpec((_BN, _D), lambda i: (i, 0))] * 2,
        out_shape=[jax.ShapeDtypeStruct((n, _D), jnp.float32)] * 2,
    )(x, wa, wb, b1)


def _edge_body(real_e, gd_ref, gs_ref, ea_ref, w1c_ref, w2_ref, b2_ref,
               g_ref, be_ref, out_ref):
    ea = ea_ref[...]
    h = gd_ref[...] + gs_ref[...] + jnp.dot(ea, w1c_ref[...], preferred_element_type=jnp.float32)
    h = _silu(h)
    h = _silu(jnp.dot(h, w2_ref[...], preferred_element_type=jnp.float32) + b2_ref[...])
    en = _ln(h, g_ref[...], be_ref[...]) + ea
    if real_e is not None:
        rid = pl.program_id(0) * _BE + lax.broadcasted_iota(jnp.int32, (_BE, 1), 0)
        en = jnp.where(rid < real_e, en, 0.0)
    out_ref[...] = en


def _edge_call(gd, gs, ea, w1c, w2, b2, g, be, real_e=None):
    e = ea.shape[0]
    full = lambda i: (0, 0)
    row = lambda i: (i, 0)
    return pl.pallas_call(
        functools.partial(_edge_body, real_e),
        grid=(e // _BE,),
        in_specs=[
            pl.BlockSpec((_BE, _D), row),
            pl.BlockSpec((_BE, _D), row),
            pl.BlockSpec((_BE, _D), row),
            pl.BlockSpec((_D, _D), full),
            pl.BlockSpec((_D, _D), full),
            pl.BlockSpec((1, _D), full),
            pl.BlockSpec((1, _D), full),
            pl.BlockSpec((1, _D), full),
        ],
        out_specs=pl.BlockSpec((_BE, _D), row),
        out_shape=jax.ShapeDtypeStruct((e, _D), jnp.float32),
    )(gd, gs, ea, w1c, w2, b2, g, be)


def _node_body_next(x_ref, p0_ref, p1_ref, p2_ref, p3_ref, wa_ref, wb_ref,
                    b1_ref, w2_ref, b2_ref, g_ref, be_ref, nwa_ref, nwb_ref,
                    nb1_ref, xo_ref, po_ref, qo_ref):
    xv = x_ref[...]
    agg = (p0_ref[...] + p1_ref[...]) + (p2_ref[...] + p3_ref[...])
    h = (jnp.dot(xv, wa_ref[...], preferred_element_type=jnp.float32)
         + jnp.dot(agg, wb_ref[...], preferred_element_type=jnp.float32)
         + b1_ref[...])
    h = _silu(h)
    h = _silu(jnp.dot(h, w2_ref[...], preferred_element_type=jnp.float32) + b2_ref[...])
    xn = _ln(h, g_ref[...], be_ref[...]) + xv
    xo_ref[...] = xn
    po_ref[...] = jnp.dot(xn, nwa_ref[...], preferred_element_type=jnp.float32) + nb1_ref[...]
    qo_ref[...] = jnp.dot(xn, nwb_ref[...], preferred_element_type=jnp.float32)


def _node_body_last(x_ref, p0_ref, p1_ref, p2_ref, p3_ref, wa_ref, wb_ref,
                    b1_ref, w2_ref, b2_ref, g_ref, be_ref, xo_ref):
    xv = x_ref[...]
    agg = (p0_ref[...] + p1_ref[...]) + (p2_ref[...] + p3_ref[...])
    h = (jnp.dot(xv, wa_ref[...], preferred_element_type=jnp.float32)
         + jnp.dot(agg, wb_ref[...], preferred_element_type=jnp.float32)
         + b1_ref[...])
    h = _silu(h)
    h = _silu(jnp.dot(h, w2_ref[...], preferred_element_type=jnp.float32) + b2_ref[...])
    xo_ref[...] = _ln(h, g_ref[...], be_ref[...]) + xv


def _node_call(x, partials_a, partials_b, wa, wb, b1, w2, b2, g, be, nxt):
    n = x.shape[0]
    nb = n // _BN
    full = lambda i: (0, 0)
    row = lambda i: (i, 0)
    vec = pl.BlockSpec((1, _D), full)
    mat = pl.BlockSpec((_D, _D), full)
    blk = pl.BlockSpec((_BN, _D), row)
    shifted = pl.BlockSpec((_BN, _D), lambda i: (i + nb, 0))
    in_specs = [
        blk,
        blk, shifted,
        blk, shifted,
        mat, mat, vec, mat, vec, vec, vec,
    ]
    args = [x, partials_a, partials_a, partials_b, partials_b,
            wa, wb, b1, w2, b2, g, be]
    if nxt is None:
        return pl.pallas_call(
            _node_body_last,
            grid=(nb,),
            in_specs=in_specs,
            out_specs=blk,
            out_shape=jax.ShapeDtypeStruct((n, _D), jnp.float32),
        )(*args)
    nwa, nwb, nb1 = nxt
    return pl.pallas_call(
        _node_body_next,
        grid=(nb,),
        in_specs=in_specs + [mat, mat, vec],
        out_specs=[blk] * 3,
        out_shape=[jax.ShapeDtypeStruct((n, _D), jnp.float32)] * 3,
    )(*args, nwa, nwb, nb1)


_NB = 3
_SNB = 3


def _gather_call(p, q, dst, src):
    n = p.shape[0]
    e = dst.shape[0]
    epw = e // _NW
    _K = _chunk_rows(epw)
    nchunk = epw // _K
    mesh = plsc.VectorSubcoreMesh(core_axis_name="c", subcore_axis_name="s")

    @functools.partial(
        pl.kernel,
        out_type=[jax.ShapeDtypeStruct((e, _D), jnp.float32)] * 2,
        mesh=mesh,
        scratch_types=[
            pltpu.VMEM((epw,), jnp.int32),
            pltpu.VMEM((epw,), jnp.int32),
            pltpu.VMEM((_NB, _K, _D), jnp.float32),
            pltpu.VMEM((_NB, _K, _D), jnp.float32),
            pltpu.SemaphoreType.DMA((_NB,)),
            pltpu.SemaphoreType.DMA((_NB,)),
            pltpu.SemaphoreType.DMA,
        ],
    )
    def k(p_hbm, q_hbm, dst_hbm, src_hbm, gd_hbm, gs_hbm,
          idxd, idxs, bufd, bufs, semg, semo, semi):
        wid = lax.axis_index("s") * 2 + lax.axis_index("c")
        base = wid * epw
        pltpu.async_copy(dst_hbm.at[pl.ds(base, epw)], idxd, semi).wait()
        pltpu.async_copy(src_hbm.at[pl.ds(base, epw)], idxs, semi).wait()

        def start_gather(c, b):
            off = c * _K
            pltpu.async_copy(p_hbm.at[idxd.at[pl.ds(off, _K)]], bufd.at[b], semg.at[b])
            pltpu.async_copy(q_hbm.at[idxs.at[pl.ds(off, _K)]], bufs.at[b], semg.at[b])

        def wait_gather(c, b):
            off = c * _K
            pltpu.make_async_copy(p_hbm.at[idxd.at[pl.ds(off, _K)]], bufd.at[b], semg.at[b]).wait()
            pltpu.make_async_copy(q_hbm.at[idxs.at[pl.ds(off, _K)]], bufs.at[b], semg.at[b]).wait()

        def start_store(c, b):
            off = base + c * _K
            pltpu.async_copy(bufd.at[b], gd_hbm.at[pl.ds(off, _K)], semo.at[b])
            pltpu.async_copy(bufs.at[b], gs_hbm.at[pl.ds(off, _K)], semo.at[b])

        def wait_store(c, b):
            off = base + c * _K
            pltpu.make_async_copy(bufd.at[b], gd_hbm.at[pl.ds(off, _K)], semo.at[b]).wait()
            pltpu.make_async_copy(bufs.at[b], gs_hbm.at[pl.ds(off, _K)], semo.at[b]).wait()

        start_gather(0, 0)

        def body(c, carry):
            b = lax.rem(c, _NB)
            nc = c + 1
            bn = lax.rem(nc, _NB)

            @pl.when(nc < nchunk)
            def _():
                @pl.when(nc >= _NB)
                def _():
                    wait_store(nc - _NB, bn)
                start_gather(nc, bn)

            wait_gather(c, b)
            start_store(c, b)
            return carry

        lax.fori_loop(0, nchunk, body, 0)
        for t in range(_NB):
            c = nchunk - _NB + t
            wait_store(c, c % _NB)

    return k(p, q, dst, src)


def _scatter_call(en, dst, n):
    e = dst.shape[0]
    epw = e // _NW
    _K = _chunk_rows(epw)
    nchunk = epw // _K
    stripe = 624
    rem = n - 16 * stripe
    zb = 16
    mesh = plsc.VectorSubcoreMesh(core_axis_name="c", subcore_axis_name="s")

    @functools.partial(
        pl.kernel,
        out_type=jax.ShapeDtypeStruct((2 * n, _D), jnp.float32),
        mesh=mesh,
        scratch_types=[
            pltpu.VMEM((_SNB, _K), jnp.int32),
            pltpu.VMEM((_SNB, _K, _D), jnp.float32),
            pltpu.VMEM((zb, _D), jnp.float32),
            pltpu.VMEM_SHARED((n, _D), jnp.float32),
            pltpu.SemaphoreType.DMA((_SNB,)),
            pltpu.SemaphoreType.DMA((_SNB,)),
            pltpu.SemaphoreType.DMA,
        ],
    )
    def k(en_hbm, dst_hbm, out_hbm, idxb, rows, zbuf, acc, semr, semw, sem):
        cid = lax.axis_index("c")
        sid = lax.axis_index("s")
        wid = sid * 2 + cid
        base = wid * epw

        zv = jnp.zeros((16,), jnp.float32)

        def zrow(r, carry):
            for j in range(_D // 16):
                zbuf[r, pl.ds(j * 16, 16)] = zv
            return carry

        lax.fori_loop(0, zb, zrow, 0)
        my_off = sid * stripe
        nzcopy = (stripe // zb) + jnp.where(sid == 15, 1, 0)

        def zcopy(t, carry):
            pltpu.async_copy(zbuf, acc.at[pl.ds(my_off + t * zb, zb)], sem).wait()
            return carry

        lax.fori_loop(0, nzcopy, zcopy, 0)
        plsc.subcore_barrier()

        def stage_and_load(c, b):
            off = base + c * _K
            pltpu.async_copy(dst_hbm.at[pl.ds(off, _K)], idxb.at[b], semr.at[b])
            pltpu.async_copy(en_hbm.at[pl.ds(off, _K)], rows.at[b], semr.at[b])

        def wait_load(c, b):
            off = base + c * _K
            pltpu.make_async_copy(dst_hbm.at[pl.ds(off, _K)], idxb.at[b], semr.at[b]).wait()
            pltpu.make_async_copy(en_hbm.at[pl.ds(off, _K)], rows.at[b], semr.at[b]).wait()

        def start_scatter(b):
            pltpu.async_copy(rows.at[b], acc.at[idxb.at[b]], semw.at[b], add=True)

        def wait_scatter(b):
            pltpu.make_async_copy(rows.at[b], acc.at[idxb.at[b]], semw.at[b]).wait()

        stage_and_load(0, 0)

        def body(c, carry):
            b = lax.rem(c, _SNB)
            nc = c + 1
            bn = lax.rem(nc, _SNB)

            @pl.when(nc < nchunk)
            def _():
                @pl.when(nc >= _SNB)
                def _():
                    wait_scatter(bn)
                stage_and_load(nc, bn)

            wait_load(c, b)
            start_scatter(b)
            return carry

        lax.fori_loop(0, nchunk, body, 0)
        for t in range(_SNB):
            wait_scatter((nchunk - _SNB + t) % _SNB)
        plsc.subcore_barrier()
        pltpu.async_copy(acc.at[pl.ds(my_off, stripe)],
                         out_hbm.at[pl.ds(cid * n + my_off, stripe)], sem).wait()
        @pl.when(sid == 15)
        def _():
            pltpu.async_copy(acc.at[pl.ds(16 * stripe, rem)],
                             out_hbm.at[pl.ds(cid * n + 16 * stripe, rem)], sem).wait()

    return k(en, dst)


def kernel(x, edge_index, edge_attr, params):
    n = x.shape[0]
    e = edge_index.shape[1]
    h = e // 2
    align = _NW * 80
    hp = -(-h // align) * align
    pad = hp - h

    def pad_idx(v):
        return jnp.concatenate([v, jnp.arange(pad, dtype=v.dtype) % n])

    src = (edge_index[0, :h], edge_index[0, h:])
    dst = (edge_index[1, :h], edge_index[1, h:])
    srcp = (pad_idx(src[0]), pad_idx(src[1]))
    dstp = (pad_idx(dst[0]), pad_idx(dst[1]))
    ea = (edge_attr[:h], edge_attr[h:])

    def split_edge(p):
        w1 = p["edge"]["W1"]
        return (w1[:_D], w1[_D:2 * _D], p["edge"]["b1"].reshape(1, _D))

    wa0, wb0, b10 = split_edge(params[0])
    pcur, qcur = _pq_call(x, wa0, wb0, b10)

    for li, p in enumerate(params):
        pe, pn = p["edge"], p["node"]
        ew = (pe["W1"][2 * _D:], pe["W2"], pe["b2"].reshape(1, _D),
              pe["g"].reshape(1, _D), pe["be"].reshape(1, _D))
        en = [None, None]
        partials = [None, None]
        for half in range(2):
            gd, gs = _gather_call(pcur, qcur, dstp[half], srcp[half])
            en[half] = _edge_call(gd, gs, ea[half], *ew)
            partials[half] = _scatter_call(en[half], dst[half], n)
        nxt = None if li == len(params) - 1 else split_edge(params[li + 1])
        res = _node_call(x, partials[0], partials[1],
                         pn["W1"][:_D], pn["W1"][_D:], pn["b1"].reshape(1, _D),
                         pn["W2"], pn["b2"].reshape(1, _D),
                         pn["g"].reshape(1, _D), pn["be"].reshape(1, _D), nxt)
        if nxt is None:
            x = res
        else:
            x, pcur, qcur = res
        ea = (en[0], en[1])
    return x
